# baseline (speedup 1.0000x reference)
import functools

import jax
import jax.numpy as jnp
from jax import lax
from jax.experimental import pallas as pl
from jax.experimental.pallas import tpu as pltpu
from jax.experimental.pallas import tpu_sc as plsc

_NC = 2
_NS = 16
_NW = _NC * _NS
_GCH = 128
_NCHUNK = 5
_UNROLL = 4
_NSPLIT = 8


def _sc_gather(table_hbm, idx_hbm, out_hbm, idx_v, rows_v, sem, *, rows_per_w):
    wid = lax.axis_index("s") * _NC + lax.axis_index("c")
    base = wid * rows_per_w
    n_ch = rows_per_w // _GCH

    def body(j, carry):
        off = base + j * _GCH
        pltpu.sync_copy(idx_hbm.at[pl.ds(off, _GCH)], idx_v)
        pltpu.async_copy(table_hbm.at[idx_v], rows_v, sem).wait()
        pltpu.sync_copy(rows_v, out_hbm.at[pl.ds(off, _GCH)])
        return carry

    lax.fori_loop(0, n_ch, body, 0)


def _embedding_gather(emb_table, idx_flat):
    n_rows = idx_flat.shape[0]
    d = emb_table.shape[1]
    assert n_rows % (_NW * _GCH) == 0
    rows_per_w = n_rows // _NW
    mesh = plsc.VectorSubcoreMesh(core_axis_name="c", subcore_axis_name="s")
    kern = pl.kernel(
        functools.partial(_sc_gather, rows_per_w=rows_per_w),
        out_type=jax.ShapeDtypeStruct((n_rows, d), jnp.float32),
        mesh=mesh,
        scratch_types=[
            pltpu.VMEM((_GCH,), jnp.int32),
            pltpu.VMEM((_GCH, d), jnp.float32),
            pltpu.SemaphoreType.DMA,
        ],
    )
    return kern(emb_table, idx_flat)


def _lstm_step(x, h, c, w_ref, u_ref, b_ref):
    nh = h.shape[1]
    z = (
        jnp.dot(x, w_ref[...], preferred_element_type=jnp.float32)
        + jnp.dot(h.astype(jnp.bfloat16), u_ref[...], preferred_element_type=jnp.float32)
        + b_ref[...]
    )
    gi = 0.5 * jnp.tanh(z[:, :nh]) + 0.5
    gf = 0.5 * jnp.tanh(z[:, nh : 2 * nh]) + 0.5
    gg = jnp.tanh(z[:, 2 * nh : 3 * nh])
    go = 0.5 * jnp.tanh(z[:, 3 * nh :]) + 0.5
    c_new = gf * c + gi * gg
    h_new = go * jnp.tanh(c_new)
    return h_new, c_new


def _run_steps(x_ref, w_ref, u_ref, b_ref, h_ref, c_ref):
    nb = h_ref.shape[0]
    sz = nb // _NSPLIT
    hs = [h_ref[j * sz : (j + 1) * sz, :] for j in range(_NSPLIT)]
    cs = [c_ref[j * sz : (j + 1) * sz, :] for j in range(_NSPLIT)]
    for i in range(_UNROLL):
        x = x_ref[i].astype(jnp.bfloat16)
        for j in range(_NSPLIT):
            hs[j], cs[j] = _lstm_step(
                x[j * sz : (j + 1) * sz, :], hs[j], cs[j], w_ref, u_ref, b_ref
            )
    for j in range(_NSPLIT):
        h_ref[j * sz : (j + 1) * sz, :] = hs[j]
        c_ref[j * sz : (j + 1) * sz, :] = cs[j]
    return jnp.concatenate(hs, axis=0)


def _lstm_mid_body(x_ref, w_ref, u_ref, b_ref, hin_ref, cin_ref, hout_ref, cout_ref):
    t = pl.program_id(0)

    @pl.when(t == 0)
    def _():
        hout_ref[...] = hin_ref[...]
        cout_ref[...] = cin_ref[...]

    _run_steps(x_ref, w_ref, u_ref, b_ref, hout_ref, cout_ref)


def _lstm_last_body(
    x_ref, w_ref, u_ref, b_ref, hin_ref, cin_ref, wd_ref, bd_ref, out_ref, h_ref, c_ref
):
    t = pl.program_id(0)

    @pl.when(t == 0)
    def _():
        h_ref[...] = hin_ref[...]
        c_ref[...] = cin_ref[...]

    h_new = _run_steps(x_ref, w_ref, u_ref, b_ref, h_ref, c_ref)

    @pl.when(t == pl.num_programs(0) - 1)
    def _():
        logits = (
            jnp.dot(h_new, wd_ref[...], preferred_element_type=jnp.float32)
            + bd_ref[...]
        )
        m = jnp.max(logits, axis=-1, keepdims=True)
        e = jnp.exp(logits - m)
        out_ref[...] = e / jnp.sum(e, axis=-1, keepdims=True)


def _wide_spec(shape):
    nd = len(shape)
    return pl.BlockSpec(shape, lambda t: (0,) * nd)


def _lstm_mid(xc, W, U, b2, h, c):
    TC_, B, D = xc.shape
    H4 = W.shape[1]
    H = H4 // 4
    return pl.pallas_call(
        _lstm_mid_body,
        grid=(TC_ // _UNROLL,),
        in_specs=[
            pl.BlockSpec((_UNROLL, B, D), lambda t: (t, 0, 0)),
            _wide_spec((D, H4)),
            _wide_spec((H, H4)),
            _wide_spec((1, H4)),
            _wide_spec((B, H)),
            _wide_spec((B, H)),
        ],
        out_specs=[_wide_spec((B, H)), _wide_spec((B, H))],
        out_shape=[
            jax.ShapeDtypeStruct((B, H), jnp.float32),
            jax.ShapeDtypeStruct((B, H), jnp.float32),
        ],
        compiler_params=pltpu.CompilerParams(dimension_semantics=("arbitrary",)),
    )(xc, W, U, b2, h, c)


def _lstm_last(xc, W, U, b2, h, c, Wdp, bdp):
    TC_, B, D = xc.shape
    H4 = W.shape[1]
    H = H4 // 4
    OP = Wdp.shape[1]
    return pl.pallas_call(
        _lstm_last_body,
        grid=(TC_ // _UNROLL,),
        in_specs=[
            pl.BlockSpec((_UNROLL, B, D), lambda t: (t, 0, 0)),
            _wide_spec((D, H4)),
            _wide_spec((H, H4)),
            _wide_spec((1, H4)),
            _wide_spec((B, H)),
            _wide_spec((B, H)),
            _wide_spec((H, OP)),
            _wide_spec((1, OP)),
        ],
        out_specs=_wide_spec((B, OP)),
        out_shape=jax.ShapeDtypeStruct((B, OP), jnp.float32),
        scratch_shapes=[
            pltpu.VMEM((B, H), jnp.float32),
            pltpu.VMEM((B, H), jnp.float32),
        ],
        compiler_params=pltpu.CompilerParams(dimension_semantics=("arbitrary",)),
    )(xc, W, U, b2, h, c, Wdp, bdp)


def kernel(inputs, emb_table, W, U, b, Wd, bd):
    B, T = inputs.shape
    D = emb_table.shape[1]
    H = U.shape[0]
    O = Wd.shape[1]
    OP = 1024
    TCH = T // _NCHUNK

    idx_flat = inputs.T.reshape(-1)
    xs = [
        _embedding_gather(
            emb_table, lax.slice(idx_flat, (k * TCH * B,), ((k + 1) * TCH * B,))
        ).reshape(TCH, B, D)
        for k in range(_NCHUNK)
    ]

    scale = jnp.concatenate(
        [
            jnp.full((2 * H,), 0.5, jnp.float32),
            jnp.ones((H,), jnp.float32),
            jnp.full((H,), 0.5, jnp.float32),
        ]
    )
    Wb = (W * scale).astype(jnp.bfloat16)
    Ub = (U * scale).astype(jnp.bfloat16)
    b2 = (b * scale).reshape(1, -1)
    Wdp = jnp.pad(Wd, ((0, 0), (0, OP - O)))
    bdp = jnp.concatenate([bd, jnp.full((OP - O,), -1e30, jnp.float32)]).reshape(1, OP)

    h = jnp.zeros((B, H), jnp.float32)
    c = jnp.zeros((B, H), jnp.float32)
    for k in range(_NCHUNK - 1):
        h, c = _lstm_mid(xs[k], Wb, Ub, b2, h, c)
    probs = _lstm_last(xs[_NCHUNK - 1], Wb, Ub, b2, h, c, Wdp, bdp)
    return probs[:, :O]


# speedup vs baseline: 1.0767x; 1.0767x over previous
import functools

import jax
import jax.numpy as jnp
from jax import lax
from jax.experimental import pallas as pl
from jax.experimental.pallas import tpu as pltpu
from jax.experimental.pallas import tpu_sc as plsc

_NC = 2
_NS = 16
_NW = _NC * _NS
_GCH = 128
_CHUNKS = (8, 16, 28, 52, 96)
_UNROLL = 4
_NSPLIT = 4


def _sc_gather(table_hbm, idx_hbm, out_hbm, idx_v, rows_v, sem, *, rows_per_w):
    wid = lax.axis_index("s") * _NC + lax.axis_index("c")
    base = wid * rows_per_w
    n_ch = rows_per_w // _GCH

    def body(j, carry):
        off = base + j * _GCH
        pltpu.sync_copy(idx_hbm.at[pl.ds(off, _GCH)], idx_v)
        pltpu.async_copy(table_hbm.at[idx_v], rows_v, sem).wait()
        pltpu.sync_copy(rows_v, out_hbm.at[pl.ds(off, _GCH)])
        return carry

    lax.fori_loop(0, n_ch, body, 0)


def _embedding_gather(emb_table, idx_flat):
    n_rows = idx_flat.shape[0]
    d = emb_table.shape[1]
    assert n_rows % (_NW * _GCH) == 0
    rows_per_w = n_rows // _NW
    mesh = plsc.VectorSubcoreMesh(core_axis_name="c", subcore_axis_name="s")
    kern = pl.kernel(
        functools.partial(_sc_gather, rows_per_w=rows_per_w),
        out_type=jax.ShapeDtypeStruct((n_rows, d), jnp.float32),
        mesh=mesh,
        scratch_types=[
            pltpu.VMEM((_GCH,), jnp.int32),
            pltpu.VMEM((_GCH, d), jnp.float32),
            pltpu.SemaphoreType.DMA,
        ],
    )
    return kern(emb_table, idx_flat)


def _lstm_step(x, h, c, w_ref, u_ref, b_ref):
    nh = h.shape[1]
    z = (
        jnp.dot(x, w_ref[...], preferred_element_type=jnp.float32)
        + jnp.dot(h.astype(jnp.bfloat16), u_ref[...], preferred_element_type=jnp.float32)
        + b_ref[...]
    )
    gi = 0.5 * jnp.tanh(z[:, :nh]) + 0.5
    gf = 0.5 * jnp.tanh(z[:, nh : 2 * nh]) + 0.5
    gg = jnp.tanh(z[:, 2 * nh : 3 * nh])
    go = 0.5 * jnp.tanh(z[:, 3 * nh :]) + 0.5
    c_new = gf * c + gi * gg
    h_new = go * jnp.tanh(c_new)
    return h_new, c_new


def _run_steps(x_ref, w_ref, u_ref, b_ref, h_ref, c_ref):
    nb = h_ref.shape[0]
    sz = nb // _NSPLIT
    hs = [h_ref[j * sz : (j + 1) * sz, :] for j in range(_NSPLIT)]
    cs = [c_ref[j * sz : (j + 1) * sz, :] for j in range(_NSPLIT)]
    for i in range(_UNROLL):
        x = x_ref[i].astype(jnp.bfloat16)
        for j in range(_NSPLIT):
            hs[j], cs[j] = _lstm_step(
                x[j * sz : (j + 1) * sz, :], hs[j], cs[j], w_ref, u_ref, b_ref
            )
    for j in range(_NSPLIT):
        h_ref[j * sz : (j + 1) * sz, :] = hs[j]
        c_ref[j * sz : (j + 1) * sz, :] = cs[j]
    return jnp.concatenate(hs, axis=0)


def _lstm_mid_body(x_ref, w_ref, u_ref, b_ref, hin_ref, cin_ref, hout_ref, cout_ref):
    t = pl.program_id(0)

    @pl.when(t == 0)
    def _():
        hout_ref[...] = hin_ref[...]
        cout_ref[...] = cin_ref[...]

    _run_steps(x_ref, w_ref, u_ref, b_ref, hout_ref, cout_ref)


def _lstm_last_body(
    x_ref, w_ref, u_ref, b_ref, hin_ref, cin_ref, wd_ref, bd_ref, out_ref, h_ref, c_ref
):
    t = pl.program_id(0)

    @pl.when(t == 0)
    def _():
        h_ref[...] = hin_ref[...]
        c_ref[...] = cin_ref[...]

    h_new = _run_steps(x_ref, w_ref, u_ref, b_ref, h_ref, c_ref)

    @pl.when(t == pl.num_programs(0) - 1)
    def _():
        logits = (
            jnp.dot(h_new, wd_ref[...], preferred_element_type=jnp.float32)
            + bd_ref[...]
        )
        m = jnp.max(logits, axis=-1, keepdims=True)
        e = jnp.exp(logits - m)
        out_ref[...] = e / jnp.sum(e, axis=-1, keepdims=True)


def _wide_spec(shape):
    nd = len(shape)
    return pl.BlockSpec(shape, lambda t: (0,) * nd)


def _lstm_mid(xc, W, U, b2, h, c):
    TC_, B, D = xc.shape
    H4 = W.shape[1]
    H = H4 // 4
    return pl.pallas_call(
        _lstm_mid_body,
        grid=(TC_ // _UNROLL,),
        in_specs=[
            pl.BlockSpec((_UNROLL, B, D), lambda t: (t, 0, 0)),
            _wide_spec((D, H4)),
            _wide_spec((H, H4)),
            _wide_spec((1, H4)),
            _wide_spec((B, H)),
            _wide_spec((B, H)),
        ],
        out_specs=[_wide_spec((B, H)), _wide_spec((B, H))],
        out_shape=[
            jax.ShapeDtypeStruct((B, H), jnp.float32),
            jax.ShapeDtypeStruct((B, H), jnp.float32),
        ],
        compiler_params=pltpu.CompilerParams(dimension_semantics=("arbitrary",)),
    )(xc, W, U, b2, h, c)


def _lstm_last(xc, W, U, b2, h, c, Wdp, bdp):
    TC_, B, D = xc.shape
    H4 = W.shape[1]
    H = H4 // 4
    OP = Wdp.shape[1]
    return pl.pallas_call(
        _lstm_last_body,
        grid=(TC_ // _UNROLL,),
        in_specs=[
            pl.BlockSpec((_UNROLL, B, D), lambda t: (t, 0, 0)),
            _wide_spec((D, H4)),
            _wide_spec((H, H4)),
            _wide_spec((1, H4)),
            _wide_spec((B, H)),
            _wide_spec((B, H)),
            _wide_spec((H, OP)),
            _wide_spec((1, OP)),
        ],
        out_specs=_wide_spec((B, OP)),
        out_shape=jax.ShapeDtypeStruct((B, OP), jnp.float32),
        scratch_shapes=[
            pltpu.VMEM((B, H), jnp.float32),
            pltpu.VMEM((B, H), jnp.float32),
        ],
        compiler_params=pltpu.CompilerParams(dimension_semantics=("arbitrary",)),
    )(xc, W, U, b2, h, c, Wdp, bdp)


def kernel(inputs, emb_table, W, U, b, Wd, bd):
    B, T = inputs.shape
    D = emb_table.shape[1]
    H = U.shape[0]
    O = Wd.shape[1]
    OP = 1024
    assert sum(_CHUNKS) == T

    idx_flat = inputs.T.reshape(-1)
    xs = []
    t0 = 0
    for tch in _CHUNKS:
        xs.append(
            _embedding_gather(
                emb_table, lax.slice(idx_flat, (t0 * B,), ((t0 + tch) * B,))
            ).reshape(tch, B, D)
        )
        t0 += tch

    scale = jnp.concatenate(
        [
            jnp.full((2 * H,), 0.5, jnp.float32),
            jnp.ones((H,), jnp.float32),
            jnp.full((H,), 0.5, jnp.float32),
        ]
    )
    Wb = (W * scale).astype(jnp.bfloat16)
    Ub = (U * scale).astype(jnp.bfloat16)
    b2 = (b * scale).reshape(1, -1)
    Wdp = jnp.pad(Wd, ((0, 0), (0, OP - O)))
    bdp = jnp.concatenate([bd, jnp.full((OP - O,), -1e30, jnp.float32)]).reshape(1, OP)

    h = jnp.zeros((B, H), jnp.float32)
    c = jnp.zeros((B, H), jnp.float32)
    for k in range(len(_CHUNKS) - 1):
        h, c = _lstm_mid(xs[k], Wb, Ub, b2, h, c)
    probs = _lstm_last(xs[-1], Wb, Ub, b2, h, c, Wdp, bdp)
    return probs[:, :O]
